# baseline (speedup 1.0000x reference)
import functools

import jax
import jax.numpy as jnp
from jax import lax
from jax.experimental import pallas as pl
from jax.experimental.pallas import tpu as pltpu
from jax.experimental.pallas import tpu_sc as plsc

N = 10000
D = 128
E = 320000
T = 5
F1 = 128
F2 = 16
HEADS1 = 8
NUM_MASKED = 4

K = 80
TILES = 32
EPT = E // TILES
NCH = EPT // K
ZR = K
NZCH = N // ZR
ZITER = (NZCH + 15) // 16
BN = 1000
BNM = 200


def _lrelu(v):
    return jnp.where(v > 0, v, 0.2 * v)


def _elu(v):
    return jnp.where(v > 0, v, jnp.exp(v) - 1.0)


def _tc_prep(x, W1, a_s, a_d):
    def body(x_ref, w_ref, as_ref, ad_ref, h_ref, ase_ref, ade_ref):
        h = jnp.dot(x_ref[...], w_ref[0], preferred_element_type=jnp.float32)
        h_ref[0] = h
        za, zd = [], []
        for hd in range(HEADS1):
            blk = h[:, hd * 16:(hd + 1) * 16]
            za.append(jnp.sum(blk * as_ref[0, hd][None, :], axis=1, keepdims=True))
            zd.append(jnp.sum(blk * ad_ref[0, hd][None, :], axis=1, keepdims=True))
        pad = jnp.zeros((BN, 8), jnp.float32)
        ase_ref[0] = jnp.concatenate(za + [pad], axis=1)
        ade_ref[0] = jnp.concatenate(zd + [pad], axis=1)

    return pl.pallas_call(
        body,
        grid=(T, N // BN),
        in_specs=[
            pl.BlockSpec((BN, D), lambda t, b: (b, 0)),
            pl.BlockSpec((1, D, F1), lambda t, b: (t, 0, 0)),
            pl.BlockSpec((1, HEADS1, 16), lambda t, b: (t, 0, 0)),
            pl.BlockSpec((1, HEADS1, 16), lambda t, b: (t, 0, 0)),
        ],
        out_specs=[
            pl.BlockSpec((1, BN, F1), lambda t, b: (t, b, 0)),
            pl.BlockSpec((1, BN, 16), lambda t, b: (t, b, 0)),
            pl.BlockSpec((1, BN, 16), lambda t, b: (t, b, 0)),
        ],
        out_shape=[
            jax.ShapeDtypeStruct((T, N, F1), jnp.float32),
            jax.ShapeDtypeStruct((T, N, 16), jnp.float32),
            jax.ShapeDtypeStruct((T, N, 16), jnp.float32),
        ],
    )(x, W1, a_s, a_d)


def _tc_edges(attr_t, dst2d):
    BE = 6400

    def body(attr_ref, dst_ref, out_ref):
        dv = dst_ref[0:1]
        rows = []
        for i in range(NUM_MASKED):
            rows.append(jnp.where(attr_ref[i:i + 1] > 0.5, dv, N))
        rows.append(dv)
        out_ref[...] = jnp.concatenate(rows, axis=0)

    return pl.pallas_call(
        body,
        grid=(E // BE,),
        in_specs=[
            pl.BlockSpec((NUM_MASKED, BE), lambda b: (0, b)),
            pl.BlockSpec((1, BE), lambda b: (0, b)),
        ],
        out_specs=pl.BlockSpec((T, BE), lambda b: (0, b)),
        out_shape=jax.ShapeDtypeStruct((T, E), jnp.int32),
    )(attr_t, dst2d)


def _make_sweep(F):
    HB = F // 16
    mesh = plsc.VectorSubcoreMesh(core_axis_name="c", subcore_axis_name="s")

    @functools.partial(
        pl.kernel,
        out_type=(
            jax.ShapeDtypeStruct((2, T, N, F), jnp.float32),
            jax.ShapeDtypeStruct((2, T, N, 16), jnp.float32),
        ),
        mesh=mesh,
        compiler_params=pltpu.CompilerParams(use_tc_tiling_on_sc=False),
        scratch_types=[
            pltpu.VMEM((2, K), jnp.int32),
            pltpu.VMEM((2, K), jnp.int32),
            pltpu.VMEM((2, K), jnp.int32),
            pltpu.VMEM((2, K), jnp.int32),
            pltpu.VMEM((2, K), jnp.int32),
            pltpu.VMEM((2, K, 16), jnp.float32),
            pltpu.VMEM((2, K, 16), jnp.float32),
            pltpu.VMEM((2, K, F), jnp.float32),
            pltpu.VMEM((2, K, 16), jnp.float32),
            pltpu.VMEM_SHARED((N + 8, F), jnp.float32),
            pltpu.VMEM_SHARED((N + 8, 16), jnp.float32),
            pltpu.SemaphoreType.DMA,
            pltpu.SemaphoreType.DMA,
            pltpu.SemaphoreType.DMA,
            pltpu.SemaphoreType.DMA,
        ],
    )
    def sweep(htab, astab, adtab, dstm, src_h, outp, denp,
              ls, ld, gb_s, gb_d, sci, asr, adr, rows, exb, acc, dacc,
              smi0, smi1, smg0, smg1):
        c = lax.axis_index("c")
        s = lax.axis_index("s")
        ebase = (c * 16 + s) * EPT
        smi = [smi0, smi1]
        smg = [smg0, smg1]

        zeros16 = jnp.zeros((16,), jnp.float32)

        for t in range(T):
            def idx_copies(j, b, t=t):
                base = ebase + j * K
                return [
                    (src_h.at[pl.ds(base, K)], ls.at[b]),
                    (dstm.at[pl.ds(t * E + base, K)], ld.at[b]),
                ]

            def fire_idx(j, b):
                for src_r, dst_r in idx_copies(j, b):
                    pltpu.async_copy(src_r, dst_r, smi[b])

            def wait_idx(j, b):
                for src_r, dst_r in idx_copies(j, b):
                    pltpu.make_async_copy(src_r, dst_r, smi[b]).wait()

            def prep_idx(b, t=t):
                for g in range(K // 16):
                    sl = pl.ds(g * 16, 16)
                    dv = ld[b, sl]
                    gb_s[b, sl] = ls[b, sl] + t * N
                    gb_d[b, sl] = dv + t * N
                    sci[b, sl] = dv

            def gather_copies(b):
                return [
                    (astab.at[gb_s.at[b]], asr.at[b]),
                    (adtab.at[gb_d.at[b]], adr.at[b]),
                    (htab.at[gb_s.at[b]], rows.at[b]),
                ]

            def fire_gather(b):
                for src_r, dst_r in gather_copies(b):
                    pltpu.async_copy(src_r, dst_r, smg[b])

            def wait_gather(b):
                for src_r, dst_r in gather_copies(b):
                    pltpu.make_async_copy(src_r, dst_r, smg[b]).wait()

            def compute(b):
                def edge(e, ecarry):
                    av = asr[b, e, :] + adr[b, e, :]
                    av = jnp.where(av > 0, av, 0.2 * av)
                    ex = jnp.exp(av)
                    if HB > 1:
                        ex = jnp.where(lax.iota(jnp.int32, 16) < 8, ex, 0.0)
                        exb[b, e, :] = ex
                        for hb in range(HB):
                            rows[b, e, pl.ds(hb * 16, 16)] = (
                                rows[b, e, pl.ds(hb * 16, 16)] * ex[hb])
                    else:
                        exb[b, e, :] = ex
                        rows[b, e, :] = rows[b, e, :] * ex
                    return ecarry

                lax.fori_loop(0, K, edge, 0)

            def scatter(b):
                pltpu.sync_copy(exb.at[b], dacc.at[sci.at[b]], add=True)
                pltpu.sync_copy(rows.at[b], acc.at[sci.at[b]], add=True)

            def phase(j, b, pg, pe):
                nb = 1 - b
                if pg:
                    wait_idx(j + 1, nb)
                    prep_idx(nb)
                    fire_gather(nb)
                if pe:
                    fire_idx(j + 2, b)
                wait_gather(b)
                compute(b)
                scatter(b)

            def zinit(r, zc):
                for hb in range(HB):
                    rows[0, r, pl.ds(hb * 16, 16)] = zeros16
                exb[0, r, :] = zeros16
                return zc

            lax.fori_loop(0, ZR, zinit, 0)

            def zrow(k, zc):
                jj = s + 16 * k

                @pl.when(jj < NZCH)
                def _():
                    pltpu.sync_copy(rows.at[0], acc.at[pl.ds(jj * ZR, ZR)])
                    pltpu.sync_copy(exb.at[0], dacc.at[pl.ds(jj * ZR, ZR)])

                return zc

            lax.fori_loop(0, ZITER, zrow, 0)
            plsc.subcore_barrier()

            fire_idx(0, 0)
            wait_idx(0, 0)
            prep_idx(0)
            fire_gather(0)
            fire_idx(1, 1)

            def step(gg, sc):
                j = 2 * gg
                phase(j, 0, pg=True, pe=True)
                phase(j + 1, 1, pg=True, pe=True)
                return sc

            lax.fori_loop(0, (NCH - 3) // 2, step, 0)
            phase(NCH - 3, 0, pg=True, pe=True)
            phase(NCH - 2, 1, pg=True, pe=False)
            phase(NCH - 1, 0, pg=False, pe=False)
            plsc.subcore_barrier()

            def drow(k, zc, t=t):
                jj = s + 16 * k

                @pl.when(jj < NZCH)
                def _():
                    pltpu.sync_copy(acc.at[pl.ds(jj * ZR, ZR)],
                                    outp.at[c, t, pl.ds(jj * ZR, ZR)])
                    pltpu.sync_copy(dacc.at[pl.ds(jj * ZR, ZR)],
                                    denp.at[c, t, pl.ds(jj * ZR, ZR)])

                return zc

            lax.fori_loop(0, ZITER, drow, 0)
            plsc.subcore_barrier()

    return sweep


_sweep128 = _make_sweep(F1)
_sweep16 = _make_sweep(F2)


def _tc_mid(outp, denp, h1, asE, adE, b1, Wf1, bf1, W2, a_s2, a_d2):
    def body(outp_ref, denp_ref, h1_ref, ase_ref, ade_ref, b1_ref, wf_ref,
             bf_ref, w2_ref, as2_ref, ad2_ref, h2_ref, as2e_ref, ad2e_ref):
        outs = []
        for t in range(T):
            num = outp_ref[0, t] + outp_ref[1, t]
            den = denp_ref[0, t] + denp_ref[1, t]
            se = jnp.exp(_lrelu(ase_ref[t] + ade_ref[t]))
            h = h1_ref[t]
            cols = []
            for hd in range(HEADS1):
                s_hd = se[:, hd:hd + 1]
                n_hd = num[:, hd * 16:(hd + 1) * 16] + s_hd * h[:, hd * 16:(hd + 1) * 16]
                d_hd = den[:, hd:hd + 1] + s_hd
                cols.append(n_hd / d_hd)
            out_t = jnp.concatenate(cols, axis=1) + b1_ref[t][None, :]
            outs.append(_elu(out_t))
        hcat = jnp.concatenate(outs, axis=1)
        hL2 = _elu(jnp.dot(hcat, wf_ref[...], preferred_element_type=jnp.float32)
                   + bf_ref[...][None, :])
        for t in range(T):
            h2 = jnp.dot(hL2, w2_ref[t], preferred_element_type=jnp.float32)
            h2_ref[t] = h2
            a2s = jnp.sum(h2 * as2_ref[t, 0][None, :], axis=1, keepdims=True)
            a2d = jnp.sum(h2 * ad2_ref[t, 0][None, :], axis=1, keepdims=True)
            as2e_ref[t] = jnp.broadcast_to(a2s, (BNM, 16))
            ad2e_ref[t] = jnp.broadcast_to(a2d, (BNM, 16))

    return pl.pallas_call(
        body,
        grid=(N // BNM,),
        in_specs=[
            pl.BlockSpec((2, T, BNM, F1), lambda b: (0, 0, b, 0)),
            pl.BlockSpec((2, T, BNM, 16), lambda b: (0, 0, b, 0)),
            pl.BlockSpec((T, BNM, F1), lambda b: (0, b, 0)),
            pl.BlockSpec((T, BNM, 16), lambda b: (0, b, 0)),
            pl.BlockSpec((T, BNM, 16), lambda b: (0, b, 0)),
            pl.BlockSpec((T, F1), lambda b: (0, 0)),
            pl.BlockSpec((T * F1, F1), lambda b: (0, 0)),
            pl.BlockSpec((F1,), lambda b: (0,)),
            pl.BlockSpec((T, F1, F2), lambda b: (0, 0, 0)),
            pl.BlockSpec((T, 1, 16), lambda b: (0, 0, 0)),
            pl.BlockSpec((T, 1, 16), lambda b: (0, 0, 0)),
        ],
        out_specs=[
            pl.BlockSpec((T, BNM, F2), lambda b: (0, b, 0)),
            pl.BlockSpec((T, BNM, 16), lambda b: (0, b, 0)),
            pl.BlockSpec((T, BNM, 16), lambda b: (0, b, 0)),
        ],
        out_shape=[
            jax.ShapeDtypeStruct((T, N, F2), jnp.float32),
            jax.ShapeDtypeStruct((T, N, 16), jnp.float32),
            jax.ShapeDtypeStruct((T, N, 16), jnp.float32),
        ],
    )(outp, denp, h1, asE, adE, b1, Wf1, bf1, W2, a_s2, a_d2)


def _tc_final(outp, denp, h2, as2e, ad2e, b2, Wf2, bf2):
    def body(outp_ref, denp_ref, h2_ref, ase_ref, ade_ref, b2_ref, wf_ref,
             bf_ref, out_ref):
        outs = []
        for t in range(T):
            num = outp_ref[0, t] + outp_ref[1, t]
            den = denp_ref[0, t] + denp_ref[1, t]
            se = jnp.exp(_lrelu(ase_ref[t] + ade_ref[t]))
            num = num + se * h2_ref[t]
            den = den + se
            out_t = num / den + b2_ref[t][None, :]
            outs.append(_elu(out_t))
        cat = jnp.concatenate(outs, axis=1)
        out_ref[...] = _elu(
            jnp.dot(cat, wf_ref[...], preferred_element_type=jnp.float32)
            + bf_ref[...][None, :])

    return pl.pallas_call(
        body,
        grid=(N // BNM,),
        in_specs=[
            pl.BlockSpec((2, T, BNM, F2), lambda b: (0, 0, b, 0)),
            pl.BlockSpec((2, T, BNM, 16), lambda b: (0, 0, b, 0)),
            pl.BlockSpec((T, BNM, F2), lambda b: (0, b, 0)),
            pl.BlockSpec((T, BNM, 16), lambda b: (0, b, 0)),
            pl.BlockSpec((T, BNM, 16), lambda b: (0, b, 0)),
            pl.BlockSpec((T, F2), lambda b: (0, 0)),
            pl.BlockSpec((T * F2, F2), lambda b: (0, 0)),
            pl.BlockSpec((F2,), lambda b: (0,)),
        ],
        out_specs=pl.BlockSpec((BNM, F2), lambda b: (b, 0)),
        out_shape=jax.ShapeDtypeStruct((N, F2), jnp.float32),
    )(outp, denp, h2, as2e, ad2e, b2, Wf2, bf2)


def kernel(x, edge_index, edge_attr, W1, a_src1, a_dst1, b1, Wf1, bf1,
           W2, a_src2, a_dst2, b2, Wf2, bf2):
    src = edge_index[0]
    dstm = _tc_edges(edge_attr.T, edge_index[1].reshape(1, E)).reshape(T * E)

    h1, asE, adE = _tc_prep(x, W1, a_src1, a_dst1)
    outp1, denp1 = _sweep128(h1.reshape(T * N, F1), asE.reshape(T * N, 16),
                             adE.reshape(T * N, 16), dstm, src)

    h2, as2e, ad2e = _tc_mid(outp1, denp1, h1, asE, adE, b1, Wf1, bf1,
                             W2, a_src2, a_dst2)
    outp2, denp2 = _sweep16(h2.reshape(T * N, F2), as2e.reshape(T * N, 16),
                            ad2e.reshape(T * N, 16), dstm, src)

    return _tc_final(outp2, denp2, h2, as2e, ad2e, b2, Wf2, bf2)


# speedup vs baseline: 1.2121x; 1.2121x over previous
import functools

import jax
import jax.numpy as jnp
from jax import lax
from jax.experimental import pallas as pl
from jax.experimental.pallas import tpu as pltpu
from jax.experimental.pallas import tpu_sc as plsc

N = 10000
D = 128
E = 320000
T = 5
F1 = 128
F2 = 16
HEADS1 = 8

K = 128
TILES = 32
EPT = 10240
NCH = EPT // K
ZR = 80
NZCH = N // ZR
ZITER = (NZCH + 15) // 16
BN = 1000
BNM = 200


def _lrelu(v):
    return jnp.where(v > 0, v, 0.2 * v)


def _elu(v):
    return jnp.where(v > 0, v, jnp.exp(v) - 1.0)


def _tc_prep(x, W1, a_s, a_d):
    def body(x_ref, w_ref, as_ref, ad_ref, h_ref, ase_ref, ade_ref):
        h = jnp.dot(x_ref[...], w_ref[0], preferred_element_type=jnp.float32)
        h_ref[0] = h
        za, zd = [], []
        for hd in range(HEADS1):
            blk = h[:, hd * 16:(hd + 1) * 16]
            za.append(jnp.sum(blk * as_ref[0, hd][None, :], axis=1, keepdims=True))
            zd.append(jnp.sum(blk * ad_ref[0, hd][None, :], axis=1, keepdims=True))
        pad = jnp.zeros((BN, 8), jnp.float32)
        ase_ref[0] = jnp.concatenate(za + [pad], axis=1)
        ade_ref[0] = jnp.concatenate(zd + [pad], axis=1)

    return pl.pallas_call(
        body,
        grid=(T, N // BN),
        in_specs=[
            pl.BlockSpec((BN, D), lambda t, b: (b, 0)),
            pl.BlockSpec((1, D, F1), lambda t, b: (t, 0, 0)),
            pl.BlockSpec((1, HEADS1, 16), lambda t, b: (t, 0, 0)),
            pl.BlockSpec((1, HEADS1, 16), lambda t, b: (t, 0, 0)),
        ],
        out_specs=[
            pl.BlockSpec((1, BN, F1), lambda t, b: (t, b, 0)),
            pl.BlockSpec((1, BN, 16), lambda t, b: (t, b, 0)),
            pl.BlockSpec((1, BN, 16), lambda t, b: (t, b, 0)),
        ],
        out_shape=[
            jax.ShapeDtypeStruct((T, N, F1), jnp.float32),
            jax.ShapeDtypeStruct((T, N, 16), jnp.float32),
            jax.ShapeDtypeStruct((T, N, 16), jnp.float32),
        ],
    )(x, W1, a_s, a_d)


def _make_sweep(F):
    HB = F // 16
    mesh = plsc.VectorSubcoreMesh(core_axis_name="c", subcore_axis_name="s")

    @functools.partial(
        pl.kernel,
        out_type=(
            jax.ShapeDtypeStruct((2, T, N, F), jnp.float32),
            jax.ShapeDtypeStruct((2, T, N, 16), jnp.float32),
        ),
        mesh=mesh,
        compiler_params=pltpu.CompilerParams(use_tc_tiling_on_sc=False),
        scratch_types=[
            pltpu.VMEM((K,), jnp.int32),
            pltpu.VMEM((K,), jnp.int32),
            pltpu.VMEM((K,), jnp.int32),
            pltpu.VMEM((K,), jnp.float32),
            pltpu.VMEM((K, 16), jnp.float32),
            pltpu.VMEM((K, 16), jnp.float32),
            pltpu.VMEM((K, F), jnp.float32),
            pltpu.VMEM((K, 16), jnp.float32),
            pltpu.VMEM((ZR, F), jnp.float32),
            pltpu.VMEM((ZR, 16), jnp.float32),
            pltpu.VMEM_SHARED((N + 8, F), jnp.float32),
            pltpu.VMEM_SHARED((N + 8, 16), jnp.float32),
            pltpu.SemaphoreType.DMA,
        ],
    )
    def sweep(h0, h1, h2, h3, h4, s0, s1, s2, s3, s4, d0, d1, d2, d3, d4,
              w0, w1, w2, w3, w4, src_h, dst_h, outp, denp,
              idx_s, idx_d, idx_dm, wmv, asr, adr, rows, exb, zbuf, zbuf16,
              acc, dacc, sem):
        c = lax.axis_index("c")
        s = lax.axis_index("s")
        ebase = (c * 16 + s) * EPT
        htabs = [h0, h1, h2, h3, h4]
        astabs = [s0, s1, s2, s3, s4]
        adtabs = [d0, d1, d2, d3, d4]
        wtabs = [w0, w1, w2, w3, w4]

        zeros16 = jnp.zeros((16,), jnp.float32)

        def zinit(r, carry):
            for hb in range(HB):
                zbuf[r, pl.ds(hb * 16, 16)] = zeros16
            zbuf16[r, :] = zeros16
            return carry

        lax.fori_loop(0, ZR, zinit, 0)

        for t in range(T):
            def zrow(k, carry):
                j = s + 16 * k

                @pl.when(j < NZCH)
                def _():
                    pltpu.sync_copy(zbuf, acc.at[pl.ds(j * ZR, ZR)])
                    pltpu.sync_copy(zbuf16, dacc.at[pl.ds(j * ZR, ZR)])

                return carry

            lax.fori_loop(0, ZITER, zrow, 0)
            plsc.subcore_barrier()

            def chunk(j, carry, t=t):
                base = ebase + j * K
                pltpu.sync_copy(src_h.at[pl.ds(base, K)], idx_s)
                pltpu.sync_copy(dst_h.at[pl.ds(base, K)], idx_d)
                pltpu.sync_copy(wtabs[t].at[pl.ds(base, K)], wmv)
                cp1 = pltpu.async_copy(astabs[t].at[idx_s], asr, sem)
                cp2 = pltpu.async_copy(adtabs[t].at[idx_d], adr, sem)
                cp3 = pltpu.async_copy(htabs[t].at[idx_s], rows, sem)
                for g in range(K // 16):
                    wv = wmv[pl.ds(g * 16, 16)]
                    dv = idx_d[pl.ds(g * 16, 16)]
                    idx_dm[pl.ds(g * 16, 16)] = jnp.where(
                        wv > 0.5, dv, jnp.full((16,), N, jnp.int32))
                cp1.wait()
                cp2.wait()
                cp3.wait()

                def edge(e, ecarry):
                    av = asr[e, :] + adr[e, :]
                    av = jnp.where(av > 0, av, 0.2 * av)
                    ex = jnp.exp(av)
                    if HB > 1:
                        ex = jnp.where(lax.iota(jnp.int32, 16) < 8, ex, 0.0)
                        exb[e, :] = ex
                        for hb in range(HB):
                            rows[e, pl.ds(hb * 16, 16)] = (
                                rows[e, pl.ds(hb * 16, 16)] * ex[hb])
                    else:
                        exb[e, :] = ex
                        rows[e, :] = rows[e, :] * ex
                    return ecarry

                lax.fori_loop(0, K, edge, 0)
                pltpu.sync_copy(exb, dacc.at[idx_dm], add=True)
                pltpu.sync_copy(rows, acc.at[idx_dm], add=True)
                return carry

            lax.fori_loop(0, NCH, chunk, 0)
            plsc.subcore_barrier()

            def drow(k, carry, t=t):
                j = s + 16 * k

                @pl.when(j < NZCH)
                def _():
                    pltpu.sync_copy(acc.at[pl.ds(j * ZR, ZR)],
                                    outp.at[c, t, pl.ds(j * ZR, ZR)])
                    pltpu.sync_copy(dacc.at[pl.ds(j * ZR, ZR)],
                                    denp.at[c, t, pl.ds(j * ZR, ZR)])

                return carry

            lax.fori_loop(0, ZITER, drow, 0)
            plsc.subcore_barrier()

    return sweep


_sweep128 = _make_sweep(F1)
_sweep16 = _make_sweep(F2)


def _tc_mid(outp, denp, h1, asE, adE, b1, Wf1, bf1, W2, a_s2, a_d2):
    def body(outp_ref, denp_ref, h1_ref, ase_ref, ade_ref, b1_ref, wf_ref,
             bf_ref, w2_ref, as2_ref, ad2_ref, h2_ref, as2e_ref, ad2e_ref):
        outs = []
        for t in range(T):
            num = outp_ref[0, t] + outp_ref[1, t]
            den = denp_ref[0, t] + denp_ref[1, t]
            se = jnp.exp(_lrelu(ase_ref[t] + ade_ref[t]))
            h = h1_ref[t]
            cols = []
            for hd in range(HEADS1):
                s_hd = se[:, hd:hd + 1]
                n_hd = num[:, hd * 16:(hd + 1) * 16] + s_hd * h[:, hd * 16:(hd + 1) * 16]
                d_hd = den[:, hd:hd + 1] + s_hd
                cols.append(n_hd / d_hd)
            out_t = jnp.concatenate(cols, axis=1) + b1_ref[t][None, :]
            outs.append(_elu(out_t))
        hcat = jnp.concatenate(outs, axis=1)
        hL2 = _elu(jnp.dot(hcat, wf_ref[...], preferred_element_type=jnp.float32)
                   + bf_ref[...][None, :])
        for t in range(T):
            h2 = jnp.dot(hL2, w2_ref[t], preferred_element_type=jnp.float32)
            h2_ref[t] = h2
            a2s = jnp.sum(h2 * as2_ref[t, 0][None, :], axis=1, keepdims=True)
            a2d = jnp.sum(h2 * ad2_ref[t, 0][None, :], axis=1, keepdims=True)
            as2e_ref[t] = jnp.broadcast_to(a2s, (BNM, 16))
            ad2e_ref[t] = jnp.broadcast_to(a2d, (BNM, 16))

    return pl.pallas_call(
        body,
        grid=(N // BNM,),
        in_specs=[
            pl.BlockSpec((2, T, BNM, F1), lambda b: (0, 0, b, 0)),
            pl.BlockSpec((2, T, BNM, 16), lambda b: (0, 0, b, 0)),
            pl.BlockSpec((T, BNM, F1), lambda b: (0, b, 0)),
            pl.BlockSpec((T, BNM, 16), lambda b: (0, b, 0)),
            pl.BlockSpec((T, BNM, 16), lambda b: (0, b, 0)),
            pl.BlockSpec((T, F1), lambda b: (0, 0)),
            pl.BlockSpec((T * F1, F1), lambda b: (0, 0)),
            pl.BlockSpec((F1,), lambda b: (0,)),
            pl.BlockSpec((T, F1, F2), lambda b: (0, 0, 0)),
            pl.BlockSpec((T, 1, 16), lambda b: (0, 0, 0)),
            pl.BlockSpec((T, 1, 16), lambda b: (0, 0, 0)),
        ],
        out_specs=[
            pl.BlockSpec((T, BNM, F2), lambda b: (0, b, 0)),
            pl.BlockSpec((T, BNM, 16), lambda b: (0, b, 0)),
            pl.BlockSpec((T, BNM, 16), lambda b: (0, b, 0)),
        ],
        out_shape=[
            jax.ShapeDtypeStruct((T, N, F2), jnp.float32),
            jax.ShapeDtypeStruct((T, N, 16), jnp.float32),
            jax.ShapeDtypeStruct((T, N, 16), jnp.float32),
        ],
    )(outp, denp, h1, asE, adE, b1, Wf1, bf1, W2, a_s2, a_d2)


def _tc_final(outp, denp, h2, as2e, ad2e, b2, Wf2, bf2):
    def body(outp_ref, denp_ref, h2_ref, ase_ref, ade_ref, b2_ref, wf_ref,
             bf_ref, out_ref):
        outs = []
        for t in range(T):
            num = outp_ref[0, t] + outp_ref[1, t]
            den = denp_ref[0, t] + denp_ref[1, t]
            se = jnp.exp(_lrelu(ase_ref[t] + ade_ref[t]))
            num = num + se * h2_ref[t]
            den = den + se
            out_t = num / den + b2_ref[t][None, :]
            outs.append(_elu(out_t))
        cat = jnp.concatenate(outs, axis=1)
        out_ref[...] = _elu(
            jnp.dot(cat, wf_ref[...], preferred_element_type=jnp.float32)
            + bf_ref[...][None, :])

    return pl.pallas_call(
        body,
        grid=(N // BNM,),
        in_specs=[
            pl.BlockSpec((2, T, BNM, F2), lambda b: (0, 0, b, 0)),
            pl.BlockSpec((2, T, BNM, 16), lambda b: (0, 0, b, 0)),
            pl.BlockSpec((T, BNM, F2), lambda b: (0, b, 0)),
            pl.BlockSpec((T, BNM, 16), lambda b: (0, b, 0)),
            pl.BlockSpec((T, BNM, 16), lambda b: (0, b, 0)),
            pl.BlockSpec((T, F2), lambda b: (0, 0)),
            pl.BlockSpec((T * F2, F2), lambda b: (0, 0)),
            pl.BlockSpec((F2,), lambda b: (0,)),
        ],
        out_specs=pl.BlockSpec((BNM, F2), lambda b: (b, 0)),
        out_shape=jax.ShapeDtypeStruct((N, F2), jnp.float32),
    )(outp, denp, h2, as2e, ad2e, b2, Wf2, bf2)


def kernel(x, edge_index, edge_attr, W1, a_src1, a_dst1, b1, Wf1, bf1,
           W2, a_src2, a_dst2, b2, Wf2, bf2):
    ept0 = E // TILES
    pad = EPT - ept0
    src = jnp.pad(edge_index[0].reshape(TILES, ept0),
                  ((0, 0), (0, pad))).reshape(-1)
    dst = jnp.pad(edge_index[1].reshape(TILES, ept0),
                  ((0, 0), (0, pad))).reshape(-1)
    wm5 = jnp.concatenate(
        [edge_attr.T, jnp.ones((1, E), jnp.float32)],
        axis=0).reshape(T, TILES, ept0)
    wmp = jnp.pad(wm5, ((0, 0), (0, 0), (0, pad))).reshape(T, TILES * EPT)
    wms = [wmp[i] for i in range(T)]

    h1, asE, adE = _tc_prep(x, W1, a_src1, a_dst1)
    h1s = [h1[i] for i in range(T)]
    ass = [asE[i] for i in range(T)]
    ads = [adE[i] for i in range(T)]
    outp1, denp1 = _sweep128(*h1s, *ass, *ads, *wms, src, dst)

    h2, as2e, ad2e = _tc_mid(outp1, denp1, h1, asE, adE, b1, Wf1, bf1,
                             W2, a_src2, a_dst2)
    h2s = [h2[i] for i in range(T)]
    as2s = [as2e[i] for i in range(T)]
    ad2s = [ad2e[i] for i in range(T)]
    outp2, denp2 = _sweep16(*h2s, *as2s, *ad2s, *wms, src, dst)

    return _tc_final(outp2, denp2, h2, as2e, ad2e, b2, Wf2, bf2)


# speedup vs baseline: 1.6945x; 1.3980x over previous
import functools

import jax
import jax.numpy as jnp
from jax import lax
from jax.experimental import pallas as pl
from jax.experimental.pallas import tpu as pltpu
from jax.experimental.pallas import tpu_sc as plsc

N = 10000
D = 128
E = 320000
T = 5
F1 = 128
F2 = 16
HEADS1 = 8

K = 80
TILES = 32
EPT = E // TILES
NCH = EPT // K
ZR = 80
NZCH = N // ZR
ZITER = (NZCH + 15) // 16
BN = 1000
BNM = 200


def _lrelu(v):
    return jnp.where(v > 0, v, 0.2 * v)


def _elu(v):
    return jnp.where(v > 0, v, jnp.exp(v) - 1.0)


def _tc_prep(x, W1, a_s, a_d):
    def body(x_ref, w_ref, as_ref, ad_ref, h_ref, ase_ref, ade_ref):
        h = jnp.dot(x_ref[...], w_ref[0], preferred_element_type=jnp.float32)
        h_ref[0] = h
        za, zd = [], []
        for hd in range(HEADS1):
            blk = h[:, hd * 16:(hd + 1) * 16]
            za.append(jnp.sum(blk * as_ref[0, hd][None, :], axis=1, keepdims=True))
            zd.append(jnp.sum(blk * ad_ref[0, hd][None, :], axis=1, keepdims=True))
        pad = jnp.zeros((BN, 8), jnp.float32)
        ase_ref[0] = jnp.concatenate(za + [pad], axis=1)
        ade_ref[0] = jnp.concatenate(zd + [pad], axis=1)

    return pl.pallas_call(
        body,
        grid=(T, N // BN),
        in_specs=[
            pl.BlockSpec((BN, D), lambda t, b: (b, 0)),
            pl.BlockSpec((1, D, F1), lambda t, b: (t, 0, 0)),
            pl.BlockSpec((1, HEADS1, 16), lambda t, b: (t, 0, 0)),
            pl.BlockSpec((1, HEADS1, 16), lambda t, b: (t, 0, 0)),
        ],
        out_specs=[
            pl.BlockSpec((1, BN, F1), lambda t, b: (t, b, 0)),
            pl.BlockSpec((1, BN, 16), lambda t, b: (t, b, 0)),
            pl.BlockSpec((1, BN, 16), lambda t, b: (t, b, 0)),
        ],
        out_shape=[
            jax.ShapeDtypeStruct((T, N, F1), jnp.float32),
            jax.ShapeDtypeStruct((T, N, 16), jnp.float32),
            jax.ShapeDtypeStruct((T, N, 16), jnp.float32),
        ],
    )(x, W1, a_s, a_d)


def _make_sweep(F):
    HB = F // 16
    mesh = plsc.VectorSubcoreMesh(core_axis_name="c", subcore_axis_name="s")

    @functools.partial(
        pl.kernel,
        out_type=(
            jax.ShapeDtypeStruct((2, T, N, F), jnp.float32),
            jax.ShapeDtypeStruct((2, T, N, 16), jnp.float32),
        ),
        mesh=mesh,
        compiler_params=pltpu.CompilerParams(use_tc_tiling_on_sc=False),
        scratch_types=[
            pltpu.VMEM((K,), jnp.int32),
            pltpu.VMEM((K,), jnp.int32),
            pltpu.VMEM((K,), jnp.int32),
            pltpu.VMEM((K,), jnp.float32),
            pltpu.VMEM((K, 16), jnp.float32),
            pltpu.VMEM((K, 16), jnp.float32),
            pltpu.VMEM((K, F), jnp.float32),
            pltpu.VMEM((K, 16), jnp.float32),
            pltpu.VMEM((ZR, F), jnp.float32),
            pltpu.VMEM((ZR, 16), jnp.float32),
            pltpu.VMEM_SHARED((N + 8, F), jnp.float32),
            pltpu.VMEM_SHARED((N + 8, 16), jnp.float32),
            pltpu.SemaphoreType.DMA,
        ],
    )
    def sweep(h0, h1, h2, h3, h4, s0, s1, s2, s3, s4, d0, d1, d2, d3, d4,
              w0, w1, w2, w3, w4, src_h, dst_h, outp, denp,
              idx_s, idx_d, idx_dm, wmv, asr, adr, rows, exb, zbuf, zbuf16,
              acc, dacc, sem):
        c = lax.axis_index("c")
        s = lax.axis_index("s")
        ebase = (c * 16 + s) * EPT
        htabs = [h0, h1, h2, h3, h4]
        astabs = [s0, s1, s2, s3, s4]
        adtabs = [d0, d1, d2, d3, d4]
        wtabs = [w0, w1, w2, w3, w4]

        zeros16 = jnp.zeros((16,), jnp.float32)

        def zinit(r, carry):
            for hb in range(HB):
                zbuf[r, pl.ds(hb * 16, 16)] = zeros16
            zbuf16[r, :] = zeros16
            return carry

        lax.fori_loop(0, ZR, zinit, 0)

        for t in range(T):
            def zrow(k, carry):
                j = s + 16 * k

                @pl.when(j < NZCH)
                def _():
                    pltpu.sync_copy(zbuf, acc.at[pl.ds(j * ZR, ZR)])
                    pltpu.sync_copy(zbuf16, dacc.at[pl.ds(j * ZR, ZR)])

                return carry

            lax.fori_loop(0, ZITER, zrow, 0)
            plsc.subcore_barrier()

            def chunk(j, carry, t=t):
                base = ebase + j * K
                pltpu.sync_copy(src_h.at[pl.ds(base, K)], idx_s)
                pltpu.sync_copy(dst_h.at[pl.ds(base, K)], idx_d)
                pltpu.sync_copy(wtabs[t].at[pl.ds(base, K)], wmv)
                cp1 = pltpu.async_copy(astabs[t].at[idx_s], asr, sem)
                cp2 = pltpu.async_copy(adtabs[t].at[idx_d], adr, sem)
                cp3 = pltpu.async_copy(htabs[t].at[idx_s], rows, sem)
                for g in range(K // 16):
                    wv = wmv[pl.ds(g * 16, 16)]
                    dv = idx_d[pl.ds(g * 16, 16)]
                    idx_dm[pl.ds(g * 16, 16)] = jnp.where(
                        wv > 0.5, dv, jnp.full((16,), N, jnp.int32))
                cp1.wait()
                cp2.wait()
                cp3.wait()

                def edge(e, ecarry):
                    av = asr[e, :] + adr[e, :]
                    av = jnp.where(av > 0, av, 0.2 * av)
                    ex = jnp.exp(av)
                    if HB > 1:
                        ex = jnp.where(lax.iota(jnp.int32, 16) < 8, ex, 0.0)
                        exb[e, :] = ex
                        for hb in range(HB):
                            rows[e, pl.ds(hb * 16, 16)] = (
                                rows[e, pl.ds(hb * 16, 16)] * ex[hb])
                    else:
                        exb[e, :] = ex
                        rows[e, :] = rows[e, :] * ex
                    return ecarry

                lax.fori_loop(0, K, edge, 0)
                pltpu.sync_copy(exb, dacc.at[idx_dm], add=True)
                pltpu.sync_copy(rows, acc.at[idx_dm], add=True)
                return carry

            lax.fori_loop(0, NCH, chunk, 0)
            plsc.subcore_barrier()

            def drow(k, carry, t=t):
                j = s + 16 * k

                @pl.when(j < NZCH)
                def _():
                    pltpu.sync_copy(acc.at[pl.ds(j * ZR, ZR)],
                                    outp.at[c, t, pl.ds(j * ZR, ZR)])
                    pltpu.sync_copy(dacc.at[pl.ds(j * ZR, ZR)],
                                    denp.at[c, t, pl.ds(j * ZR, ZR)])

                return carry

            lax.fori_loop(0, ZITER, drow, 0)
            plsc.subcore_barrier()

    return sweep


_sweep128 = _make_sweep(F1)

FW = T * F2
K2 = 40
EPT0 = E // TILES
NCH2 = EPT0 // K2
ZR2 = K2
NZ2 = N // ZR2
ZI2 = (NZ2 + 15) // 16

_mesh2 = plsc.VectorSubcoreMesh(core_axis_name="c", subcore_axis_name="s")


@functools.partial(
    pl.kernel,
    out_type=(
        jax.ShapeDtypeStruct((2, N, FW), jnp.float32),
        jax.ShapeDtypeStruct((2, N, FW), jnp.float32),
    ),
    mesh=_mesh2,
    compiler_params=pltpu.CompilerParams(use_tc_tiling_on_sc=False),
    scratch_types=[
        pltpu.VMEM((K2,), jnp.int32),
        pltpu.VMEM((K2,), jnp.int32),
        pltpu.VMEM((K2, FW), jnp.float32),
        pltpu.VMEM((K2, FW), jnp.float32),
        pltpu.VMEM((K2, FW), jnp.float32),
        pltpu.VMEM((K2, FW), jnp.float32),
        pltpu.VMEM((K2, FW), jnp.float32),
        pltpu.VMEM((ZR2, FW), jnp.float32),
        pltpu.VMEM_SHARED((N, FW), jnp.float32),
        pltpu.VMEM_SHARED((N, FW), jnp.float32),
        pltpu.SemaphoreType.DMA,
    ],
)
def _sweep_l2(h2c, as2c, ad2c, wmxh, src_h, dst_h, outp, denp,
              idx_s, idx_d, wmxb, asr, adr, rows, exb, zb, acc, dacc, sem):
    c = lax.axis_index("c")
    s = lax.axis_index("s")
    ebase = (c * 16 + s) * EPT0
    zeros16 = jnp.zeros((16,), jnp.float32)

    def zinit(r, carry):
        for t5 in range(T):
            zb[r, pl.ds(t5 * 16, 16)] = zeros16
        return carry

    lax.fori_loop(0, ZR2, zinit, 0)

    def zrow(k, carry):
        jj = s + 16 * k

        @pl.when(jj < NZ2)
        def _():
            pltpu.sync_copy(zb, acc.at[pl.ds(jj * ZR2, ZR2)])
            pltpu.sync_copy(zb, dacc.at[pl.ds(jj * ZR2, ZR2)])

        return carry

    lax.fori_loop(0, ZI2, zrow, 0)
    plsc.subcore_barrier()

    def chunk(j, carry):
        base = ebase + j * K2
        pltpu.sync_copy(src_h.at[pl.ds(base, K2)], idx_s)
        pltpu.sync_copy(dst_h.at[pl.ds(base, K2)], idx_d)
        cp0 = pltpu.async_copy(wmxh.at[pl.ds(base, K2)], wmxb, sem)
        cp1 = pltpu.async_copy(as2c.at[idx_s], asr, sem)
        cp2 = pltpu.async_copy(ad2c.at[idx_d], adr, sem)
        cp3 = pltpu.async_copy(h2c.at[idx_s], rows, sem)
        cp0.wait()
        cp1.wait()
        cp2.wait()
        cp3.wait()

        def edge(e, ecarry):
            for t5 in range(T):
                sl = pl.ds(t5 * 16, 16)
                av = asr[e, sl] + adr[e, sl]
                av = jnp.where(av > 0, av, 0.2 * av)
                ex = jnp.exp(av) * wmxb[e, sl]
                exb[e, sl] = ex
                rows[e, sl] = rows[e, sl] * ex
            return ecarry

        lax.fori_loop(0, K2, edge, 0)
        pltpu.sync_copy(exb, dacc.at[idx_d], add=True)
        pltpu.sync_copy(rows, acc.at[idx_d], add=True)
        return carry

    lax.fori_loop(0, NCH2, chunk, 0)
    plsc.subcore_barrier()

    def drow(k, carry):
        jj = s + 16 * k

        @pl.when(jj < NZ2)
        def _():
            pltpu.sync_copy(acc.at[pl.ds(jj * ZR2, ZR2)],
                            outp.at[c, pl.ds(jj * ZR2, ZR2)])
            pltpu.sync_copy(dacc.at[pl.ds(jj * ZR2, ZR2)],
                            denp.at[c, pl.ds(jj * ZR2, ZR2)])

        return carry

    lax.fori_loop(0, ZI2, drow, 0)
    plsc.subcore_barrier()


def _tc_mid(outp, denp, h1, asE, adE, b1, Wf1, bf1, W2, a_s2, a_d2):
    def body(outp_ref, denp_ref, h1_ref, ase_ref, ade_ref, b1_ref, wf_ref,
             bf_ref, w2_ref, as2_ref, ad2_ref, h2_ref, as2e_ref, ad2e_ref):
        outs = []
        for t in range(T):
            num = outp_ref[0, t] + outp_ref[1, t]
            den = denp_ref[0, t] + denp_ref[1, t]
            se = jnp.exp(_lrelu(ase_ref[t] + ade_ref[t]))
            h = h1_ref[t]
            cols = []
            for hd in range(HEADS1):
                s_hd = se[:, hd:hd + 1]
                n_hd = num[:, hd * 16:(hd + 1) * 16] + s_hd * h[:, hd * 16:(hd + 1) * 16]
                d_hd = den[:, hd:hd + 1] + s_hd
                cols.append(n_hd / d_hd)
            out_t = jnp.concatenate(cols, axis=1) + b1_ref[t][None, :]
            outs.append(_elu(out_t))
        hcat = jnp.concatenate(outs, axis=1)
        hL2 = _elu(jnp.dot(hcat, wf_ref[...], preferred_element_type=jnp.float32)
                   + bf_ref[...][None, :])
        h2cols, ascols, adcols = [], [], []
        for t in range(T):
            h2 = jnp.dot(hL2, w2_ref[t], preferred_element_type=jnp.float32)
            h2cols.append(h2)
            a2s = jnp.sum(h2 * as2_ref[t, 0][None, :], axis=1, keepdims=True)
            a2d = jnp.sum(h2 * ad2_ref[t, 0][None, :], axis=1, keepdims=True)
            ascols.append(jnp.broadcast_to(a2s, (BNM, 16)))
            adcols.append(jnp.broadcast_to(a2d, (BNM, 16)))
        h2_ref[...] = jnp.concatenate(h2cols, axis=1)
        as2e_ref[...] = jnp.concatenate(ascols, axis=1)
        ad2e_ref[...] = jnp.concatenate(adcols, axis=1)

    return pl.pallas_call(
        body,
        grid=(N // BNM,),
        in_specs=[
            pl.BlockSpec((2, T, BNM, F1), lambda b: (0, 0, b, 0)),
            pl.BlockSpec((2, T, BNM, 16), lambda b: (0, 0, b, 0)),
            pl.BlockSpec((T, BNM, F1), lambda b: (0, b, 0)),
            pl.BlockSpec((T, BNM, 16), lambda b: (0, b, 0)),
            pl.BlockSpec((T, BNM, 16), lambda b: (0, b, 0)),
            pl.BlockSpec((T, F1), lambda b: (0, 0)),
            pl.BlockSpec((T * F1, F1), lambda b: (0, 0)),
            pl.BlockSpec((F1,), lambda b: (0,)),
            pl.BlockSpec((T, F1, F2), lambda b: (0, 0, 0)),
            pl.BlockSpec((T, 1, 16), lambda b: (0, 0, 0)),
            pl.BlockSpec((T, 1, 16), lambda b: (0, 0, 0)),
        ],
        out_specs=[
            pl.BlockSpec((BNM, T * F2), lambda b: (b, 0)),
            pl.BlockSpec((BNM, T * 16), lambda b: (b, 0)),
            pl.BlockSpec((BNM, T * 16), lambda b: (b, 0)),
        ],
        out_shape=[
            jax.ShapeDtypeStruct((N, T * F2), jnp.float32),
            jax.ShapeDtypeStruct((N, T * 16), jnp.float32),
            jax.ShapeDtypeStruct((N, T * 16), jnp.float32),
        ],
    )(outp, denp, h1, asE, adE, b1, Wf1, bf1, W2, a_s2, a_d2)


def _tc_final(outp, denp, h2, as2e, ad2e, b2, Wf2, bf2):
    def body(outp_ref, denp_ref, h2_ref, ase_ref, ade_ref, b2_ref, wf_ref,
             bf_ref, out_ref):
        outs = []
        for t in range(T):
            sl = slice(t * 16, (t + 1) * 16)
            num = outp_ref[0, :, sl] + outp_ref[1, :, sl]
            den = denp_ref[0, :, sl] + denp_ref[1, :, sl]
            se = jnp.exp(_lrelu(ase_ref[:, sl] + ade_ref[:, sl]))
            num = num + se * h2_ref[:, sl]
            den = den + se
            out_t = num / den + b2_ref[t][None, :]
            outs.append(_elu(out_t))
        cat = jnp.concatenate(outs, axis=1)
        out_ref[...] = _elu(
            jnp.dot(cat, wf_ref[...], preferred_element_type=jnp.float32)
            + bf_ref[...][None, :])

    return pl.pallas_call(
        body,
        grid=(N // BNM,),
        in_specs=[
            pl.BlockSpec((2, BNM, T * F2), lambda b: (0, b, 0)),
            pl.BlockSpec((2, BNM, T * F2), lambda b: (0, b, 0)),
            pl.BlockSpec((BNM, T * F2), lambda b: (b, 0)),
            pl.BlockSpec((BNM, T * 16), lambda b: (b, 0)),
            pl.BlockSpec((BNM, T * 16), lambda b: (b, 0)),
            pl.BlockSpec((T, F2), lambda b: (0, 0)),
            pl.BlockSpec((T * F2, F2), lambda b: (0, 0)),
            pl.BlockSpec((F2,), lambda b: (0,)),
        ],
        out_specs=pl.BlockSpec((BNM, F2), lambda b: (b, 0)),
        out_shape=jax.ShapeDtypeStruct((N, F2), jnp.float32),
    )(outp, denp, h2, as2e, ad2e, b2, Wf2, bf2)


def kernel(x, edge_index, edge_attr, W1, a_src1, a_dst1, b1, Wf1, bf1,
           W2, a_src2, a_dst2, b2, Wf2, bf2):
    src = edge_index[0]
    dst = edge_index[1]
    wms = [edge_attr[:, i] for i in range(4)] + [jnp.ones((E,), jnp.float32)]

    h1, asE, adE = _tc_prep(x, W1, a_src1, a_dst1)
    h1s = [h1[i] for i in range(T)]
    ass = [asE[i] for i in range(T)]
    ads = [adE[i] for i in range(T)]
    outp1, denp1 = _sweep128(*h1s, *ass, *ads, *wms, src, dst)

    h2c, as2c, ad2c = _tc_mid(outp1, denp1, h1, asE, adE, b1, Wf1, bf1,
                              W2, a_src2, a_dst2)
    wmx = jnp.broadcast_to(
        jnp.concatenate([edge_attr, jnp.ones((E, 1), jnp.float32)],
                        axis=1)[:, :, None],
        (E, T, 16)).reshape(E, T * 16)
    outp2, denp2 = _sweep_l2(h2c, as2c, ad2c, wmx, src, dst)

    return _tc_final(outp2, denp2, h2c, as2c, ad2c, b2, Wf2, bf2)
